# baseline (speedup 1.0000x reference)
import functools

import jax
import jax.numpy as jnp
from jax import lax
from jax.experimental import pallas as pl
from jax.experimental.pallas import tpu as pltpu
from jax.experimental.pallas import tpu_sc as plsc

N = 10000
E = 320000
F = 128
FH = F // 2
NCORES = 2
NSUB = 16
NW = NCORES * NSUB
EPW = E // NW
AEC = 120
ACHUNKS = -(-EPW // AEC)
EPAD = ACHUNKS * AEC - EPW
EPWP = ACHUNKS * AEC
NPAD = 10240
ROWS_PER_SUB = NPAD // NSUB
ZROWS = 128

_SC_MESH = dict(core_axis_name="c", subcore_axis_name="s")


@functools.partial(
    pl.kernel,
    out_type=jax.ShapeDtypeStruct((NCORES, NPAD, 16), jnp.float32),
    mesh=plsc.VectorSubcoreMesh(**_SC_MESH),
    scratch_types=[
        pltpu.VMEM((ACHUNKS, AEC), jnp.int32),
        pltpu.VMEM((AEC, 16), jnp.float32),
        pltpu.VMEM((ZROWS, 16), jnp.float32),
        pltpu.VMEM_SHARED((NPAD, 16), jnp.float32),
    ],
    compiler_params=pltpu.CompilerParams(use_tc_tiling_on_sc=False),
)
def _deg_kernel(dst_hbm, out_hbm, idx_v, ones_v, zero_v, shared):
    c = lax.axis_index("c")
    s = lax.axis_index("s")

    def fill(i, _):
        ones_v[i, :] = jnp.ones((16,), jnp.float32)
        return 0

    lax.fori_loop(0, AEC, fill, 0)

    def zfill(i, _):
        zero_v[i, :] = jnp.zeros((16,), jnp.float32)
        return 0

    lax.fori_loop(0, ZROWS, zfill, 0)

    for t in range(ROWS_PER_SUB // ZROWS):
        pltpu.sync_copy(zero_v, shared.at[pl.ds(s * ROWS_PER_SUB + t * ZROWS, ZROWS)])
    plsc.subcore_barrier()

    pltpu.sync_copy(dst_hbm.at[c * NSUB + s], idx_v)

    def body(j, _):
        pltpu.sync_copy(ones_v, shared.at[idx_v.at[j]], add=True)
        return 0

    lax.fori_loop(0, ACHUNKS, body, 0)
    plsc.subcore_barrier()

    for t in range(ROWS_PER_SUB // ZROWS):
        r0 = s * ROWS_PER_SUB + t * ZROWS
        pltpu.sync_copy(shared.at[pl.ds(r0, ZROWS)], out_hbm.at[c, pl.ds(r0, ZROWS)])


@functools.partial(
    pl.kernel,
    out_type=jax.ShapeDtypeStruct((NCORES, 2, NPAD, FH), jnp.float32),
    mesh=plsc.VectorSubcoreMesh(**_SC_MESH),
    scratch_types=[
        pltpu.VMEM((EPWP,), jnp.int32),
        pltpu.VMEM((ACHUNKS, AEC), jnp.int32),
        [pltpu.VMEM((AEC, FH), jnp.float32) for _ in range(4)],
        pltpu.VMEM((ZROWS, FH), jnp.float32),
        [pltpu.SemaphoreType.DMA for _ in range(4)],
        pltpu.VMEM_SHARED((NPAD, FH), jnp.float32),
    ],
    compiler_params=pltpu.CompilerParams(use_tc_tiling_on_sc=False),
)
def _agg_kernel(ha_hbm, hb_hbm, src_hbm, dst_hbm, out_hbm,
                idxs_v, idxd_v, rows_bufs, zero_v, sems, shared):
    c = lax.axis_index("c")
    s = lax.axis_index("s")

    def zfill(i, _):
        zero_v[i // (FH // 16), pl.ds((i % (FH // 16)) * 16, 16)] = jnp.zeros(
            (16,), jnp.float32
        )
        return 0

    lax.fori_loop(0, ZROWS * (FH // 16), zfill, 0)

    wid = c * NSUB + s
    pltpu.sync_copy(src_hbm.at[pl.ds(wid * EPWP, EPWP)], idxs_v)
    pltpu.sync_copy(dst_hbm.at[wid], idxd_v)

    for h, h_hbm in enumerate((ha_hbm, hb_hbm)):
        for t in range(ROWS_PER_SUB // ZROWS):
            pltpu.sync_copy(zero_v, shared.at[pl.ds(s * ROWS_PER_SUB + t * ZROWS, ZROWS)])
        plsc.subcore_barrier()

        def gidx(k):
            return idxs_v.at[pl.ds(k * AEC, AEC)]

        for b in range(3):
            pltpu.async_copy(h_hbm.at[gidx(b)], rows_bufs[b], sems[b])

        def step(k, b):
            pltpu.make_async_copy(h_hbm.at[gidx(k)], rows_bufs[b], sems[b]).wait()
            nb = (b + 3) % 4

            @pl.when(k + 3 < ACHUNKS)
            def _():
                pltpu.async_copy(h_hbm.at[gidx(k + 3)], rows_bufs[nb], sems[nb])

            pltpu.sync_copy(rows_bufs[b], shared.at[idxd_v.at[k]], add=True)

        def body(j, _):
            for b in range(4):
                step(4 * j + b, b)
            return 0

        lax.fori_loop(0, ACHUNKS // 4, body, 0)
        plsc.subcore_barrier()

        for t in range(ROWS_PER_SUB // ZROWS):
            r0 = s * ROWS_PER_SUB + t * ZROWS
            pltpu.sync_copy(shared.at[pl.ds(r0, ZROWS)], out_hbm.at[c, h, pl.ds(r0, ZROWS)])


_R = 1000


def _stage_a_body(degp_ref, x_ref, w1_ref, ha_ref, hb_ref, dinv_ref):
    deg = 1.0 + degp_ref[0] + degp_ref[1]
    dinv = lax.rsqrt(deg)
    dinv_ref[...] = dinv
    h = jnp.dot(x_ref[...], w1_ref[...], preferred_element_type=jnp.float32)
    hs = h * dinv[:, :1]
    ha_ref[...] = hs[:, :FH]
    hb_ref[...] = hs[:, FH:]


def _stage_b_body(p_ref, ha_ref, hb_ref, dinv_ref, w2_ref, b1_ref,
                  ha2_ref, hb2_ref):
    dinv = dinv_ref[...][:, :1]
    sa = (p_ref[0, 0] + p_ref[1, 0] + ha_ref[...]) * dinv + b1_ref[..., :FH]
    sb = (p_ref[0, 1] + p_ref[1, 1] + hb_ref[...]) * dinv + b1_ref[..., FH:]
    h1a = jnp.maximum(sa, 0.0)
    h1b = jnp.maximum(sb, 0.0)
    hs2 = (
        jnp.dot(h1a, w2_ref[:FH, :], preferred_element_type=jnp.float32)
        + jnp.dot(h1b, w2_ref[FH:, :], preferred_element_type=jnp.float32)
    ) * dinv
    ha2_ref[...] = hs2[:, :FH]
    hb2_ref[...] = hs2[:, FH:]


def _stage_c_body(p_ref, ha_ref, hb_ref, dinv_ref, wfc_ref, b2_ref, bfc_ref,
                  out_ref):
    dinv = dinv_ref[...][:, :1]
    oa = (p_ref[0, 0] + p_ref[1, 0] + ha_ref[...]) * dinv + b2_ref[..., :FH]
    ob = (p_ref[0, 1] + p_ref[1, 1] + hb_ref[...]) * dinv + b2_ref[..., FH:]
    logits = (
        jnp.dot(oa, wfc_ref[:FH, :], preferred_element_type=jnp.float32)
        + jnp.dot(ob, wfc_ref[FH:, :], preferred_element_type=jnp.float32)
        + bfc_ref[...]
    )
    m = jnp.max(logits, axis=1, keepdims=True)
    lse = m + jnp.log(jnp.sum(jnp.exp(logits - m), axis=1, keepdims=True))
    out_ref[...] = logits - lse


def _full(block_shape):
    return pl.BlockSpec(block_shape, lambda i: tuple(0 for _ in block_shape))


def _rows(block_shape, dim=0):
    def imap(i):
        return tuple(i if d == dim else 0 for d in range(len(block_shape)))

    return pl.BlockSpec(block_shape, imap)


_stage_a = pl.pallas_call(
    _stage_a_body,
    grid=(N // _R,),
    in_specs=[_rows((NCORES, _R, 16), dim=1), _rows((_R, F)), _full((F, F))],
    out_specs=[_rows((_R, FH)), _rows((_R, FH)), _rows((_R, 16))],
    out_shape=[
        jax.ShapeDtypeStruct((N, FH), jnp.float32),
        jax.ShapeDtypeStruct((N, FH), jnp.float32),
        jax.ShapeDtypeStruct((N, 16), jnp.float32),
    ],
)

_stage_b = pl.pallas_call(
    _stage_b_body,
    grid=(N // _R,),
    in_specs=[
        _rows((NCORES, 2, _R, FH), dim=2),
        _rows((_R, FH)),
        _rows((_R, FH)),
        _rows((_R, 16)),
        _full((F, F)),
        _full((1, F)),
    ],
    out_specs=[_rows((_R, FH)), _rows((_R, FH))],
    out_shape=[
        jax.ShapeDtypeStruct((N, FH), jnp.float32),
        jax.ShapeDtypeStruct((N, FH), jnp.float32),
    ],
)


def _make_stage_c(ncls):
    return pl.pallas_call(
        _stage_c_body,
        grid=(N // _R,),
        in_specs=[
            _rows((NCORES, 2, _R, FH), dim=2),
            _rows((_R, FH)),
            _rows((_R, FH)),
            _rows((_R, 16)),
            _full((F, ncls)),
            _full((1, F)),
            _full((1, ncls)),
        ],
        out_specs=_rows((_R, ncls)),
        out_shape=jax.ShapeDtypeStruct((N, ncls), jnp.float32),
    )


def kernel(x, edge_index, W1, b1, W2, b2, Wfc, bfc):
    assert x.shape == (N, F) and edge_index.shape == (2, E)
    ei = edge_index.astype(jnp.int32)
    src_w = ei[0].reshape(NW, EPW)
    dst_w = ei[1].reshape(NW, EPW)
    zpad = jnp.zeros((NW, EPAD), jnp.int32)

    def chunked(a, pad_val):
        padded = jnp.concatenate([a, zpad + pad_val], axis=1)
        return padded.reshape(NW, ACHUNKS, AEC)

    src = jnp.concatenate([src_w, zpad], axis=1).reshape(NW * EPWP)
    dst = chunked(dst_w, N + jnp.arange(NW, dtype=jnp.int32)[:, None])

    degp = _deg_kernel(dst)
    ha1, hb1, dinv16 = _stage_a(degp, x, W1)
    p1 = _agg_kernel(ha1, hb1, src, dst)
    ha2, hb2 = _stage_b(p1, ha1, hb1, dinv16, W2, b1.reshape(1, F))
    p2 = _agg_kernel(ha2, hb2, src, dst)
    return _make_stage_c(Wfc.shape[1])(
        p2, ha2, hb2, dinv16, Wfc, b2.reshape(1, F), bfc.reshape(1, -1)
    )


# speedup vs baseline: 1.5316x; 1.5316x over previous
import functools

import jax
import jax.numpy as jnp
from jax import lax
from jax.experimental import pallas as pl
from jax.experimental.pallas import tpu as pltpu
from jax.experimental.pallas import tpu_sc as plsc

N = 10000
E = 320000
F = 128
FH = F // 2
NCORES = 2
NSUB = 16
NW = NCORES * NSUB
EPW = E // NW
AEC = 125
ACHUNKS = EPW // AEC
NPAD = 10240
ROWS_PER_SUB = NPAD // NSUB
ZROWS = 128

_SC_MESH = dict(core_axis_name="c", subcore_axis_name="s")


@functools.partial(
    pl.kernel,
    out_type=jax.ShapeDtypeStruct((NCORES, NPAD, 16), jnp.float32),
    mesh=plsc.VectorSubcoreMesh(**_SC_MESH),
    scratch_types=[
        pltpu.VMEM((ACHUNKS, AEC), jnp.int32),
        pltpu.VMEM((AEC, 16), jnp.float32),
        pltpu.VMEM((ZROWS, 16), jnp.float32),
        pltpu.VMEM_SHARED((NPAD, 16), jnp.float32),
    ],
    compiler_params=pltpu.CompilerParams(use_tc_tiling_on_sc=False),
)
def _deg_kernel(dst_hbm, out_hbm, idx_v, ones_v, zero_v, shared):
    c = lax.axis_index("c")
    s = lax.axis_index("s")

    def fill(i, _):
        ones_v[i, :] = jnp.ones((16,), jnp.float32)
        return 0

    lax.fori_loop(0, AEC, fill, 0)

    def zfill(i, _):
        zero_v[i, :] = jnp.zeros((16,), jnp.float32)
        return 0

    lax.fori_loop(0, ZROWS, zfill, 0)

    for t in range(ROWS_PER_SUB // ZROWS):
        pltpu.sync_copy(zero_v, shared.at[pl.ds(s * ROWS_PER_SUB + t * ZROWS, ZROWS)])
    plsc.subcore_barrier()

    pltpu.sync_copy(dst_hbm.at[c * NSUB + s], idx_v)

    def body(j, _):
        pltpu.sync_copy(ones_v, shared.at[idx_v.at[j]], add=True)
        return 0

    lax.fori_loop(0, ACHUNKS, body, 0)
    plsc.subcore_barrier()

    for t in range(ROWS_PER_SUB // ZROWS):
        r0 = s * ROWS_PER_SUB + t * ZROWS
        pltpu.sync_copy(shared.at[pl.ds(r0, ZROWS)], out_hbm.at[c, pl.ds(r0, ZROWS)])


@functools.partial(
    pl.kernel,
    out_type=jax.ShapeDtypeStruct((NCORES, 2, NPAD, FH), jnp.float32),
    mesh=plsc.VectorSubcoreMesh(**_SC_MESH),
    scratch_types=[
        pltpu.VMEM((ACHUNKS, AEC), jnp.int32),
        pltpu.VMEM((ACHUNKS, AEC), jnp.int32),
        [pltpu.VMEM((AEC, FH), jnp.float32) for _ in range(4)],
        pltpu.VMEM((ZROWS, FH), jnp.float32),
        [pltpu.SemaphoreType.DMA for _ in range(4)],
        pltpu.VMEM_SHARED((NPAD, FH), jnp.float32),
    ],
    compiler_params=pltpu.CompilerParams(use_tc_tiling_on_sc=False),
)
def _agg_kernel(ha_hbm, hb_hbm, src_hbm, dst_hbm, out_hbm,
                idxs_v, idxd_v, rows_bufs, zero_v, sems, shared):
    c = lax.axis_index("c")
    s = lax.axis_index("s")

    def zfill(i, _):
        zero_v[i // (FH // 16), pl.ds((i % (FH // 16)) * 16, 16)] = jnp.zeros(
            (16,), jnp.float32
        )
        return 0

    lax.fori_loop(0, ZROWS * (FH // 16), zfill, 0)

    wid = c * NSUB + s
    pltpu.sync_copy(src_hbm.at[wid], idxs_v)
    pltpu.sync_copy(dst_hbm.at[wid], idxd_v)

    for h, h_hbm in enumerate((ha_hbm, hb_hbm)):
        for t in range(ROWS_PER_SUB // ZROWS):
            pltpu.sync_copy(zero_v, shared.at[pl.ds(s * ROWS_PER_SUB + t * ZROWS, ZROWS)])
        plsc.subcore_barrier()

        def gidx(k):
            return idxs_v.at[k]

        for b in range(3):
            pltpu.async_copy(h_hbm.at[gidx(b)], rows_bufs[b], sems[b])

        def step(k, b):
            pltpu.make_async_copy(h_hbm.at[gidx(k)], rows_bufs[b], sems[b]).wait()
            nb = (b + 3) % 4

            @pl.when(k + 3 < ACHUNKS)
            def _():
                pltpu.async_copy(h_hbm.at[gidx(k + 3)], rows_bufs[nb], sems[nb])

            pltpu.sync_copy(rows_bufs[b], shared.at[idxd_v.at[k]], add=True)

        def body(j, _):
            for b in range(4):
                step(4 * j + b, b)
            return 0

        lax.fori_loop(0, ACHUNKS // 4, body, 0)
        plsc.subcore_barrier()

        for t in range(ROWS_PER_SUB // ZROWS):
            r0 = s * ROWS_PER_SUB + t * ZROWS
            pltpu.sync_copy(shared.at[pl.ds(r0, ZROWS)], out_hbm.at[c, h, pl.ds(r0, ZROWS)])


_R = 1000


def _stage_a_body(degp_ref, x_ref, w1_ref, ha_ref, hb_ref, dinv_ref):
    deg = 1.0 + degp_ref[0] + degp_ref[1]
    dinv = lax.rsqrt(deg)
    dinv_ref[...] = dinv
    h = jnp.dot(x_ref[...], w1_ref[...], preferred_element_type=jnp.float32)
    hs = h * dinv[:, :1]
    ha_ref[...] = hs[:, :FH]
    hb_ref[...] = hs[:, FH:]


def _stage_b_body(p_ref, ha_ref, hb_ref, dinv_ref, w2_ref, b1_ref,
                  ha2_ref, hb2_ref):
    dinv = dinv_ref[...][:, :1]
    sa = (p_ref[0, 0] + p_ref[1, 0] + ha_ref[...]) * dinv + b1_ref[..., :FH]
    sb = (p_ref[0, 1] + p_ref[1, 1] + hb_ref[...]) * dinv + b1_ref[..., FH:]
    h1a = jnp.maximum(sa, 0.0)
    h1b = jnp.maximum(sb, 0.0)
    hs2 = (
        jnp.dot(h1a, w2_ref[:FH, :], preferred_element_type=jnp.float32)
        + jnp.dot(h1b, w2_ref[FH:, :], preferred_element_type=jnp.float32)
    ) * dinv
    ha2_ref[...] = hs2[:, :FH]
    hb2_ref[...] = hs2[:, FH:]


def _stage_c_body(p_ref, ha_ref, hb_ref, dinv_ref, wfc_ref, b2_ref, bfc_ref,
                  out_ref):
    dinv = dinv_ref[...][:, :1]
    oa = (p_ref[0, 0] + p_ref[1, 0] + ha_ref[...]) * dinv + b2_ref[..., :FH]
    ob = (p_ref[0, 1] + p_ref[1, 1] + hb_ref[...]) * dinv + b2_ref[..., FH:]
    logits = (
        jnp.dot(oa, wfc_ref[:FH, :], preferred_element_type=jnp.float32)
        + jnp.dot(ob, wfc_ref[FH:, :], preferred_element_type=jnp.float32)
        + bfc_ref[...]
    )
    m = jnp.max(logits, axis=1, keepdims=True)
    lse = m + jnp.log(jnp.sum(jnp.exp(logits - m), axis=1, keepdims=True))
    out_ref[...] = logits - lse


def _full(block_shape):
    return pl.BlockSpec(block_shape, lambda i: tuple(0 for _ in block_shape))


def _rows(block_shape, dim=0):
    def imap(i):
        return tuple(i if d == dim else 0 for d in range(len(block_shape)))

    return pl.BlockSpec(block_shape, imap)


_stage_a = pl.pallas_call(
    _stage_a_body,
    grid=(N // _R,),
    in_specs=[_rows((NCORES, _R, 16), dim=1), _rows((_R, F)), _full((F, F))],
    out_specs=[_rows((_R, FH)), _rows((_R, FH)), _rows((_R, 16))],
    out_shape=[
        jax.ShapeDtypeStruct((N, FH), jnp.float32),
        jax.ShapeDtypeStruct((N, FH), jnp.float32),
        jax.ShapeDtypeStruct((N, 16), jnp.float32),
    ],
)

_stage_b = pl.pallas_call(
    _stage_b_body,
    grid=(N // _R,),
    in_specs=[
        _rows((NCORES, 2, _R, FH), dim=2),
        _rows((_R, FH)),
        _rows((_R, FH)),
        _rows((_R, 16)),
        _full((F, F)),
        _full((1, F)),
    ],
    out_specs=[_rows((_R, FH)), _rows((_R, FH))],
    out_shape=[
        jax.ShapeDtypeStruct((N, FH), jnp.float32),
        jax.ShapeDtypeStruct((N, FH), jnp.float32),
    ],
)


def _make_stage_c(ncls):
    return pl.pallas_call(
        _stage_c_body,
        grid=(N // _R,),
        in_specs=[
            _rows((NCORES, 2, _R, FH), dim=2),
            _rows((_R, FH)),
            _rows((_R, FH)),
            _rows((_R, 16)),
            _full((F, ncls)),
            _full((1, F)),
            _full((1, ncls)),
        ],
        out_specs=_rows((_R, ncls)),
        out_shape=jax.ShapeDtypeStruct((N, ncls), jnp.float32),
    )


def kernel(x, edge_index, W1, b1, W2, b2, Wfc, bfc):
    assert x.shape == (N, F) and edge_index.shape == (2, E)
    ei = edge_index.astype(jnp.int32)
    src = ei[0].reshape(NW, ACHUNKS, AEC)
    dst = ei[1].reshape(NW, ACHUNKS, AEC)

    degp = _deg_kernel(dst)
    ha1, hb1, dinv16 = _stage_a(degp, x, W1)
    p1 = _agg_kernel(ha1, hb1, src, dst)
    ha2, hb2 = _stage_b(p1, ha1, hb1, dinv16, W2, b1.reshape(1, F))
    p2 = _agg_kernel(ha2, hb2, src, dst)
    return _make_stage_c(Wfc.shape[1])(
        p2, ha2, hb2, dinv16, Wfc, b2.reshape(1, F), bfc.reshape(1, -1)
    )


# speedup vs baseline: 1.5554x; 1.0155x over previous
import functools

import jax
import jax.numpy as jnp
from jax import lax
from jax.experimental import pallas as pl
from jax.experimental.pallas import tpu as pltpu
from jax.experimental.pallas import tpu_sc as plsc

N = 10000
E = 320000
F = 128
FH = F // 2
NCORES = 2
NSUB = 16
NW = NCORES * NSUB
EPW = E // NW
AEC = 125
ACHUNKS = EPW // AEC
NPAD = 10240
ROWS_PER_SUB = NPAD // NSUB
ZROWS = 128

_SC_MESH = dict(core_axis_name="c", subcore_axis_name="s")


@functools.partial(
    pl.kernel,
    out_type=jax.ShapeDtypeStruct((NCORES, NPAD, 16), jnp.float32),
    mesh=plsc.VectorSubcoreMesh(**_SC_MESH),
    scratch_types=[
        pltpu.VMEM((ACHUNKS, AEC), jnp.int32),
        pltpu.VMEM((AEC, 16), jnp.float32),
        pltpu.VMEM((ZROWS, 16), jnp.float32),
        pltpu.VMEM_SHARED((NPAD, 16), jnp.float32),
    ],
    compiler_params=pltpu.CompilerParams(use_tc_tiling_on_sc=False),
)
def _deg_kernel(dst_hbm, out_hbm, idx_v, ones_v, zero_v, shared):
    c = lax.axis_index("c")
    s = lax.axis_index("s")

    def fill(i, _):
        ones_v[i, :] = jnp.ones((16,), jnp.float32)
        return 0

    lax.fori_loop(0, AEC, fill, 0)

    def zfill(i, _):
        zero_v[i, :] = jnp.zeros((16,), jnp.float32)
        return 0

    lax.fori_loop(0, ZROWS, zfill, 0)

    for t in range(ROWS_PER_SUB // ZROWS):
        pltpu.sync_copy(zero_v, shared.at[pl.ds(s * ROWS_PER_SUB + t * ZROWS, ZROWS)])
    plsc.subcore_barrier()

    pltpu.sync_copy(dst_hbm.at[c * NSUB + s], idx_v)

    def body(j, _):
        pltpu.sync_copy(ones_v, shared.at[idx_v.at[j]], add=True)
        return 0

    lax.fori_loop(0, ACHUNKS, body, 0)
    plsc.subcore_barrier()

    for t in range(ROWS_PER_SUB // ZROWS):
        r0 = s * ROWS_PER_SUB + t * ZROWS
        pltpu.sync_copy(shared.at[pl.ds(r0, ZROWS)], out_hbm.at[c, pl.ds(r0, ZROWS)])


@functools.partial(
    pl.kernel,
    out_type=jax.ShapeDtypeStruct((NCORES, 2, NPAD, FH), jnp.float32),
    mesh=plsc.VectorSubcoreMesh(**_SC_MESH),
    scratch_types=[
        pltpu.VMEM((ACHUNKS, AEC), jnp.int32),
        pltpu.VMEM((ACHUNKS, AEC), jnp.int32),
        [pltpu.VMEM((AEC, FH), jnp.float32) for _ in range(4)],
        pltpu.VMEM((ZROWS, FH), jnp.float32),
        [pltpu.SemaphoreType.DMA for _ in range(4)],
        pltpu.VMEM_SHARED((NPAD, FH), jnp.float32),
    ],
    compiler_params=pltpu.CompilerParams(use_tc_tiling_on_sc=False),
)
def _agg_kernel(ha_hbm, hb_hbm, src_hbm, dst_hbm, out_hbm,
                idxs_v, idxd_v, rows_bufs, zero_v, sems, shared):
    c = lax.axis_index("c")
    s = lax.axis_index("s")

    def zfill(i, _):
        zero_v[i // (FH // 16), pl.ds((i % (FH // 16)) * 16, 16)] = jnp.zeros(
            (16,), jnp.float32
        )
        return 0

    lax.fori_loop(0, ZROWS * (FH // 16), zfill, 0)

    wid = c * NSUB + s
    pltpu.sync_copy(src_hbm.at[wid], idxs_v)
    pltpu.sync_copy(dst_hbm.at[wid], idxd_v)

    for h, h_hbm in enumerate((ha_hbm, hb_hbm)):
        for t in range(ROWS_PER_SUB // ZROWS):
            pltpu.sync_copy(zero_v, shared.at[pl.ds(s * ROWS_PER_SUB + t * ZROWS, ZROWS)])
        plsc.subcore_barrier()

        def gidx(k):
            return idxs_v.at[k]

        for b in range(3):
            pltpu.async_copy(h_hbm.at[gidx(b)], rows_bufs[b], sems[b])

        def step(k, b):
            pltpu.make_async_copy(h_hbm.at[gidx(k)], rows_bufs[b], sems[b]).wait()
            nb = (b + 3) % 4

            @pl.when(k + 3 < ACHUNKS)
            def _():
                pltpu.async_copy(h_hbm.at[gidx(k + 3)], rows_bufs[nb], sems[nb])

            pltpu.sync_copy(rows_bufs[b], shared.at[idxd_v.at[k]], add=True)

        def body(j, _):
            for b in range(4):
                step(4 * j + b, b)
            return 0

        lax.fori_loop(0, ACHUNKS // 4, body, 0)
        plsc.subcore_barrier()

        for t in range(ROWS_PER_SUB // ZROWS):
            r0 = s * ROWS_PER_SUB + t * ZROWS
            pltpu.sync_copy(shared.at[pl.ds(r0, ZROWS)], out_hbm.at[c, h, pl.ds(r0, ZROWS)])


_R = 2000


def _stage_a_body(degp_ref, x_ref, w1_ref, ha_ref, hb_ref, dinv_ref):
    deg = 1.0 + degp_ref[0] + degp_ref[1]
    dinv = lax.rsqrt(deg)
    dinv_ref[...] = dinv
    h = jnp.dot(x_ref[...], w1_ref[...], preferred_element_type=jnp.float32)
    hs = h * dinv[:, :1]
    ha_ref[...] = hs[:, :FH]
    hb_ref[...] = hs[:, FH:]


def _stage_b_body(p_ref, ha_ref, hb_ref, dinv_ref, w2_ref, b1_ref,
                  ha2_ref, hb2_ref):
    dinv = dinv_ref[...][:, :1]
    sa = (p_ref[0, 0] + p_ref[1, 0] + ha_ref[...]) * dinv + b1_ref[..., :FH]
    sb = (p_ref[0, 1] + p_ref[1, 1] + hb_ref[...]) * dinv + b1_ref[..., FH:]
    h1a = jnp.maximum(sa, 0.0)
    h1b = jnp.maximum(sb, 0.0)
    hs2 = (
        jnp.dot(h1a, w2_ref[:FH, :], preferred_element_type=jnp.float32)
        + jnp.dot(h1b, w2_ref[FH:, :], preferred_element_type=jnp.float32)
    ) * dinv
    ha2_ref[...] = hs2[:, :FH]
    hb2_ref[...] = hs2[:, FH:]


def _stage_c_body(p_ref, ha_ref, hb_ref, dinv_ref, wfc_ref, b2_ref, bfc_ref,
                  out_ref):
    dinv = dinv_ref[...][:, :1]
    oa = (p_ref[0, 0] + p_ref[1, 0] + ha_ref[...]) * dinv + b2_ref[..., :FH]
    ob = (p_ref[0, 1] + p_ref[1, 1] + hb_ref[...]) * dinv + b2_ref[..., FH:]
    logits = (
        jnp.dot(oa, wfc_ref[:FH, :], preferred_element_type=jnp.float32)
        + jnp.dot(ob, wfc_ref[FH:, :], preferred_element_type=jnp.float32)
        + bfc_ref[...]
    )
    m = jnp.max(logits, axis=1, keepdims=True)
    lse = m + jnp.log(jnp.sum(jnp.exp(logits - m), axis=1, keepdims=True))
    out_ref[...] = logits - lse


def _full(block_shape):
    return pl.BlockSpec(block_shape, lambda i: tuple(0 for _ in block_shape))


def _rows(block_shape, dim=0):
    def imap(i):
        return tuple(i if d == dim else 0 for d in range(len(block_shape)))

    return pl.BlockSpec(block_shape, imap)


_stage_a = pl.pallas_call(
    _stage_a_body,
    grid=(N // _R,),
    in_specs=[_rows((NCORES, _R, 16), dim=1), _rows((_R, F)), _full((F, F))],
    out_specs=[_rows((_R, FH)), _rows((_R, FH)), _rows((_R, 16))],
    out_shape=[
        jax.ShapeDtypeStruct((N, FH), jnp.float32),
        jax.ShapeDtypeStruct((N, FH), jnp.float32),
        jax.ShapeDtypeStruct((N, 16), jnp.float32),
    ],
)

_stage_b = pl.pallas_call(
    _stage_b_body,
    grid=(N // _R,),
    in_specs=[
        _rows((NCORES, 2, _R, FH), dim=2),
        _rows((_R, FH)),
        _rows((_R, FH)),
        _rows((_R, 16)),
        _full((F, F)),
        _full((1, F)),
    ],
    out_specs=[_rows((_R, FH)), _rows((_R, FH))],
    out_shape=[
        jax.ShapeDtypeStruct((N, FH), jnp.float32),
        jax.ShapeDtypeStruct((N, FH), jnp.float32),
    ],
)


def _make_stage_c(ncls):
    return pl.pallas_call(
        _stage_c_body,
        grid=(N // _R,),
        in_specs=[
            _rows((NCORES, 2, _R, FH), dim=2),
            _rows((_R, FH)),
            _rows((_R, FH)),
            _rows((_R, 16)),
            _full((F, ncls)),
            _full((1, F)),
            _full((1, ncls)),
        ],
        out_specs=_rows((_R, ncls)),
        out_shape=jax.ShapeDtypeStruct((N, ncls), jnp.float32),
    )


def kernel(x, edge_index, W1, b1, W2, b2, Wfc, bfc):
    assert x.shape == (N, F) and edge_index.shape == (2, E)
    ei = edge_index.astype(jnp.int32)
    src = ei[0].reshape(NW, ACHUNKS, AEC)
    dst = ei[1].reshape(NW, ACHUNKS, AEC)

    degp = _deg_kernel(dst)
    ha1, hb1, dinv16 = _stage_a(degp, x, W1)
    p1 = _agg_kernel(ha1, hb1, src, dst)
    ha2, hb2 = _stage_b(p1, ha1, hb1, dinv16, W2, b1.reshape(1, F))
    p2 = _agg_kernel(ha2, hb2, src, dst)
    return _make_stage_c(Wfc.shape[1])(
        p2, ha2, hb2, dinv16, Wfc, b2.reshape(1, F), bfc.reshape(1, -1)
    )
